# Initial kernel scaffold; baseline (speedup 1.0000x reference)
import jax
import jax.numpy as jnp
from jax.experimental import pallas as pl


def kernel(cls_score, pred_object, true_label, true_object, iter_num):
    raise NotImplementedError("write your pallas kernel here")


# speedup vs baseline: 2.0466x; 2.0466x over previous
import functools

import jax
import jax.numpy as jnp
import numpy as np
from jax import lax
from jax.experimental import pallas as pl
from jax.experimental.pallas import tpu as pltpu
from jax.experimental.pallas import tpu_sc as plsc

B = 16
W = 64
H = 64
A = 5
C = 80
M = 100
ANCHORS = np.array([[1.19, 1.98], [2.79, 4.59], [4.53, 8.92], [8.06, 5.29], [10.32, 10.65]], dtype=np.float32)
IOU_THR = 0.5
SCALE_NOOBJ = 0.5
SCALE_OBJ = 5.0
ANCHOR_TRAIN_ITERS = 12000

MPAD = 128
HALF = 64
NW = 32
ROWS = W * H
PP = 32
DROWS = ROWS // 2
L = 16


def _splat_i(v):
    return jnp.full((L,), v, dtype=jnp.int32)


def _sigmoid(x):
    return 1.0 / (1.0 + jnp.exp(-x))


def _sc_body(cls_hbm, pred_hbm, lab_hbm, tob_hbm, out_hbm,
             tob_v, cells_v, idx_v, clsrows_v, predrows_v, lab_v, msk_v,
             dense_v, acc_v, sem0, sem1, sem2, sem3):
    nc = 2
    wid = lax.axis_index("s") * nc + lax.axis_index("c")
    b = wid // 2
    h = wid % 2
    iota = lax.iota(jnp.int32, L)
    zf = jnp.zeros((L,), jnp.float32)

    dstart = b * ROWS + h * DROWS
    dense_cp = pltpu.make_async_copy(pred_hbm.at[pl.ds(dstart, DROWS)], dense_v, sem0)
    dense_cp.start()

    pltpu.sync_copy(tob_hbm.at[b], tob_v)
    lab_cp = pltpu.make_async_copy(lab_hbm.at[b, pl.ds(h * HALF, HALF)], lab_v, sem1)
    lab_cp.start()

    for k in range(MPAD // L):
        tx = tob_v[0, pl.ds(k * L, L)] * jnp.float32(W)
        ty = tob_v[1, pl.ds(k * L, L)] * jnp.float32(H)
        ci = jnp.minimum(jnp.maximum(tx.astype(jnp.int32), 0), W - 1)
        cj = jnp.minimum(jnp.maximum(ty.astype(jnp.int32), 0), H - 1)
        mglob = iota + k * L
        cell = jnp.where(mglob < M, ci * H + cj, -1 - mglob)
        cells_v[pl.ds(k * L, L)] = cell

    for kl in range(HALF // L):
        cell = plsc.load_gather(cells_v, [iota + (h * HALF + kl * L)])
        row = jnp.where(cell >= 0, cell + b * ROWS, 0)
        idx_v[pl.ds(kl * L, L)] = row

    cls_cp = pltpu.make_async_copy(cls_hbm.at[idx_v], clsrows_v, sem2)
    cls_cp.start()
    pred_cp = pltpu.make_async_copy(pred_hbm.at[idx_v], predrows_v, sem3)
    pred_cp.start()

    dense_cp.wait()

    def dense_body(r, dn_acc):
        rows = iota + r * L
        for a in range(A):
            p4 = plsc.load_gather(dense_v, [rows, _splat_i(a * 5 + 4)])
            s = _sigmoid(p4)
            dn_acc = dn_acc + s * s
        return dn_acc

    dense_acc = lax.fori_loop(0, DROWS // L, dense_body, zf)

    pred_cp.wait()

    corr_acc = zf
    obj_acc = zf
    prior_acc = zf
    true_acc = zf

    for cloc in range(HALF // L):
        kk = h * (HALF // L) + cloc
        m_vec = iota + kk * L
        mycells = plsc.load_gather(cells_v, [m_vec])

        def loser_body(t, loser):
            j = t // L
            s = t - j * L
            rot_idx = (iota + s) & (L - 1)
            rot_cells = plsc.load_gather(cells_v, [rot_idx + j * L])
            mp_vec = rot_idx + j * L
            hit = (rot_cells == mycells) & (mp_vec > m_vec)
            return loser | hit.astype(jnp.int32)

        loser = lax.fori_loop(0, (MPAD // L) * L, loser_body, jnp.zeros((L,), jnp.int32))
        winner = (loser == 0) & (m_vec < M)

        base = h * HALF + cloc * L
        tx = tob_v[0, pl.ds(base, L)] * jnp.float32(W)
        ty = tob_v[1, pl.ds(base, L)] * jnp.float32(H)
        tw = tob_v[2, pl.ds(base, L)] * jnp.float32(W)
        th = tob_v[3, pl.ds(base, L)] * jnp.float32(H)
        ci_f = jnp.minimum(jnp.maximum(tx.astype(jnp.int32), 0), W - 1).astype(jnp.float32)
        cj_f = jnp.minimum(jnp.maximum(ty.astype(jnp.int32), 0), H - 1).astype(jnp.float32)
        rows = iota + cloc * L

        for a in range(A):
            aw = float(ANCHORS[a, 0])
            ah = float(ANCHORS[a, 1])
            inter = jnp.minimum(tw, aw) * jnp.minimum(th, ah)
            union = tw * th + (aw * ah) - inter
            iou = inter / (union + 1e-9)
            maskb = winner & (iou > IOU_THR)
            maskf = jnp.where(maskb, 1.0, 0.0).astype(jnp.float32)
            msk_v[a, pl.ds(cloc * L, L)] = maskf

            p0 = plsc.load_gather(predrows_v, [rows, _splat_i(a * 5 + 0)])
            p1 = plsc.load_gather(predrows_v, [rows, _splat_i(a * 5 + 1)])
            p2 = plsc.load_gather(predrows_v, [rows, _splat_i(a * 5 + 2)])
            p3 = plsc.load_gather(predrows_v, [rows, _splat_i(a * 5 + 3)])
            p4 = plsc.load_gather(predrows_v, [rows, _splat_i(a * 5 + 4)])

            obj = _sigmoid(p4)
            corr_acc = corr_acc + jnp.where(maskb, obj * obj, zf)
            d0 = obj - iou
            obj_acc = obj_acc + jnp.where(maskb, d0 * d0, zf)
            wx = jnp.exp(p2) * aw
            wy = jnp.exp(p3) * ah
            dpx = wx - aw
            dpy = wy - ah
            prior_acc = prior_acc + jnp.where(maskb, dpx * dpx + dpy * dpy, zf)
            bx = _sigmoid(p0) + ci_f
            by = _sigmoid(p1) + cj_f
            d1 = bx - tx
            d2 = by - ty
            d3 = wx - tw
            d4 = wy - th
            true_acc = true_acc + jnp.where(maskb, d1 * d1 + d2 * d2 + d3 * d3 + d4 * d4, zf)

    cls_cp.wait()
    lab_cp.wait()

    def score_body(m, sc_acc):
        labs = [plsc.load_gather(lab_v, [_splat_i(m), iota + k * L]) for k in range(C // L)]
        for a in range(A):
            ma = plsc.load_gather(msk_v, [_splat_i(a), _splat_i(m)])
            sacc = zf
            for k in range(C // L):
                d = plsc.load_gather(clsrows_v, [_splat_i(m), iota + (a * C + k * L)]) - labs[k]
                sacc = sacc + d * d
            sc_acc = sc_acc + ma * sacc
        return sc_acc

    score_acc = lax.fori_loop(0, HALF, score_body, zf)

    acc_v[0] = corr_acc
    acc_v[1] = obj_acc
    acc_v[2] = prior_acc
    acc_v[3] = true_acc
    acc_v[4] = score_acc
    acc_v[5] = dense_acc
    acc_v[6] = zf
    acc_v[7] = zf
    pltpu.sync_copy(acc_v, out_hbm.at[wid])


@jax.jit
def _yolo_loss_sc(cls2, pred2, labp, tobp):
    mesh = plsc.VectorSubcoreMesh(core_axis_name="c", subcore_axis_name="s")
    run = functools.partial(
        pl.kernel,
        mesh=mesh,
        compiler_params=pltpu.CompilerParams(
            needs_layout_passes=False, use_tc_tiling_on_sc=False),
        out_type=jax.ShapeDtypeStruct((NW, 8, L), jnp.float32),
        scratch_types=[
            pltpu.VMEM((4, MPAD), jnp.float32),
            pltpu.VMEM((MPAD,), jnp.int32),
            pltpu.VMEM((HALF,), jnp.int32),
            pltpu.VMEM((HALF, A * C), jnp.float32),
            pltpu.VMEM((HALF, PP), jnp.float32),
            pltpu.VMEM((HALF, C), jnp.float32),
            pltpu.VMEM((A, HALF), jnp.float32),
            pltpu.VMEM((DROWS, PP), jnp.float32),
            pltpu.VMEM((8, L), jnp.float32),
            pltpu.SemaphoreType.DMA,
            pltpu.SemaphoreType.DMA,
            pltpu.SemaphoreType.DMA,
            pltpu.SemaphoreType.DMA,
        ],
    )(_sc_body)
    return run(cls2, pred2, labp, tobp)


def kernel(cls_score, pred_object, true_label, true_object, iter_num):
    cls2 = cls_score.reshape(B * W * H, A * C)
    pred2 = jnp.pad(pred_object.reshape(B * W * H, A * 5), ((0, 0), (0, PP - A * 5)))
    labp = jnp.pad(true_label, ((0, 0), (0, MPAD - M), (0, 0)))
    tobp = jnp.pad(jnp.transpose(true_object, (0, 2, 1)), ((0, 0), (0, 0), (0, MPAD - M)))

    out = _yolo_loss_sc(cls2, pred2, labp, tobp)
    sums = jnp.sum(out, axis=(0, 2))
    corr, obj_s, prior_s, true_s, score_s, dense_s = (sums[i] for i in range(6))

    n1 = float(B * W * H * A)
    need_prior = (iter_num < ANCHOR_TRAIN_ITERS).astype(jnp.float32)
    noobj_loss = SCALE_NOOBJ * 0.5 * (dense_s - corr) / n1
    obj_loss = SCALE_OBJ * 0.5 * obj_s / n1
    prior_loss = need_prior * SCALE_OBJ * 0.5 * prior_s / (n1 * 2)
    true_loss = SCALE_OBJ * 0.5 * true_s / (n1 * 4)
    score_loss = SCALE_OBJ * 0.5 * score_s / (n1 * C)
    return (noobj_loss + obj_loss + prior_loss + true_loss + score_loss) / 4.0


# speedup vs baseline: 8.1236x; 3.9693x over previous
import functools

import jax
import jax.numpy as jnp
import numpy as np
from jax import lax
from jax.experimental import pallas as pl
from jax.experimental.pallas import tpu as pltpu
from jax.experimental.pallas import tpu_sc as plsc

B = 16
W = 64
H = 64
A = 5
C = 80
M = 100
ANCHORS = np.array([[1.19, 1.98], [2.79, 4.59], [4.53, 8.92], [8.06, 5.29], [10.32, 10.65]], dtype=np.float32)
IOU_THR = 0.5
SCALE_NOOBJ = 0.5
SCALE_OBJ = 5.0
ANCHOR_TRAIN_ITERS = 12000

MPAD = 128
HALF = 64
NW = 32
ROWS = W * H
PP = 32
L = 16
CLS_ROWS = B * W * A * H
PLANES = B * A * 5


def _splat_i(v):
    return jnp.full((L,), v, dtype=jnp.int32)


def _sigmoid(x):
    return 1.0 / (1.0 + jnp.exp(-x))


def _sc_body(clsT, pred4, predT2, labF, tobF, out_hbm,
             tob_v, cells_v, base_v, idxp_v, sub_v, clsbuf,
             predrows, planes_v, lab_v, msk_v, acc_v,
             semg, semb, semp, seml):
    nc = 2
    wid = lax.axis_index("s") * nc + lax.axis_index("c")
    b = wid // 2
    hh = wid % 2
    iota = lax.iota(jnp.int32, L)
    zf = jnp.zeros((L,), jnp.float32)

    plane_cps = []
    for a in range(A):
        src = predT2.at[pl.ds(pl.multiple_of(((b * A + a) * 5 + 4) * W + hh * 32, 32), 32), :]
        cp = pltpu.make_async_copy(src, planes_v.at[pl.ds(a * 32, 32), :], semp)
        cp.start()
        plane_cps.append(cp)

    pltpu.sync_copy(tobF.at[pl.ds(pl.multiple_of(b * 512, 512), 512)], tob_v)
    lab_cp = pltpu.make_async_copy(
        labF.at[pl.ds(pl.multiple_of(b * (MPAD * C) + hh * (HALF * C), HALF * C), HALF * C)], lab_v, seml)
    lab_cp.start()

    for k in range(MPAD // L):
        tx = tob_v[pl.ds(k * L, L)] * jnp.float32(W)
        ty = tob_v[pl.ds(MPAD + k * L, L)] * jnp.float32(H)
        cw = jnp.minimum(jnp.maximum(tx.astype(jnp.int32), 0), W - 1)
        ch = jnp.minimum(jnp.maximum(ty.astype(jnp.int32), 0), H - 1)
        mglob = iota + k * L
        cell = jnp.where(mglob < M, cw * H + ch, -1 - mglob)
        cells_v[pl.ds(k * L, L)] = cell

    for kl in range(HALF // L):
        base = pl.multiple_of(hh * HALF + kl * L, L)
        tx = tob_v[pl.ds(base, L)] * jnp.float32(W)
        ty = tob_v[pl.ds(MPAD + base, L)] * jnp.float32(H)
        cw = jnp.minimum(jnp.maximum(tx.astype(jnp.int32), 0), W - 1)
        ch = jnp.minimum(jnp.maximum(ty.astype(jnp.int32), 0), H - 1)
        base_v[pl.ds(kl * L, L)] = (b * (W * A * H) + cw * (A * H)) + ch
        base_v[pl.ds(HALF + kl * L, L)] = b * ROWS + cw * H + ch

    for kl in range(HALF // L):
        r32 = plsc.load_gather(base_v, [iota + (HALF + kl * L)])
        idxp_v[pl.ds(kl * L, L)] = r32 >> 2
        sub_v[pl.ds(kl * L, L)] = (r32 & 3) * PP

    pred_cp = pltpu.make_async_copy(pred4.at[idxp_v], predrows, semb)
    pred_cp.start()

    def fire(m, _):
        cb = jnp.max(plsc.load_gather(base_v, [_splat_i(0) + m]))
        for a in range(A):
            pltpu.make_async_copy(
                clsT.at[pl.ds(cb + a * H, 1), :],
                clsbuf.at[pl.ds(m * A + a, 1), :], semg).start()
        return 0

    lax.fori_loop(0, HALF, fire, 0)

    pred_cp.wait()

    corr_acc = zf
    obj_acc = zf
    prior_acc = zf
    true_acc = zf

    for cloc in range(HALF // L):
        kk = hh * (HALF // L) + cloc
        m_vec = iota + kk * L
        mycells = plsc.load_gather(cells_v, [m_vec])

        def loser_body(t, loser):
            j = t // L
            s = t - j * L
            rot_idx = (iota + s) & (L - 1)
            rot_cells = plsc.load_gather(cells_v, [rot_idx + j * L])
            mp_vec = rot_idx + j * L
            hit = (rot_cells == mycells) & (mp_vec > m_vec)
            return loser | hit.astype(jnp.int32)

        loser = lax.fori_loop(0, (MPAD // L) * L, loser_body, jnp.zeros((L,), jnp.int32))
        winner = (loser == 0) & (m_vec < M)

        base = pl.multiple_of(hh * HALF + cloc * L, L)
        tx = tob_v[pl.ds(base, L)] * jnp.float32(W)
        ty = tob_v[pl.ds(MPAD + base, L)] * jnp.float32(H)
        tw = tob_v[pl.ds(2 * MPAD + base, L)] * jnp.float32(W)
        th = tob_v[pl.ds(3 * MPAD + base, L)] * jnp.float32(H)
        ci_f = jnp.minimum(jnp.maximum(tx.astype(jnp.int32), 0), W - 1).astype(jnp.float32)
        cj_f = jnp.minimum(jnp.maximum(ty.astype(jnp.int32), 0), H - 1).astype(jnp.float32)
        mrow = iota + cloc * L
        scol = plsc.load_gather(sub_v, [mrow])

        for a in range(A):
            aw = float(ANCHORS[a, 0])
            ah = float(ANCHORS[a, 1])
            inter = jnp.minimum(tw, aw) * jnp.minimum(th, ah)
            union = tw * th + (aw * ah) - inter
            iou = inter / (union + 1e-9)
            maskb = winner & (iou > IOU_THR)
            maskf = jnp.where(maskb, 1.0, 0.0).astype(jnp.float32)
            msk_v[pl.ds(a * HALF + cloc * L, L)] = maskf

            p0 = plsc.load_gather(predrows, [mrow, scol + (a * 5 + 0)])
            p1 = plsc.load_gather(predrows, [mrow, scol + (a * 5 + 1)])
            p2 = plsc.load_gather(predrows, [mrow, scol + (a * 5 + 2)])
            p3 = plsc.load_gather(predrows, [mrow, scol + (a * 5 + 3)])
            p4 = plsc.load_gather(predrows, [mrow, scol + (a * 5 + 4)])

            obj = _sigmoid(p4)
            corr_acc = corr_acc + jnp.where(maskb, obj * obj, zf)
            d0 = obj - iou
            obj_acc = obj_acc + jnp.where(maskb, d0 * d0, zf)
            wx = jnp.exp(p2) * aw
            wy = jnp.exp(p3) * ah
            dpx = wx - aw
            dpy = wy - ah
            prior_acc = prior_acc + jnp.where(maskb, dpx * dpx + dpy * dpy, zf)
            bx = _sigmoid(p0) + ci_f
            by = _sigmoid(p1) + cj_f
            d1 = bx - tx
            d2 = by - ty
            d3 = wx - tw
            d4 = wy - th
            true_acc = true_acc + jnp.where(maskb, d1 * d1 + d2 * d2 + d3 * d3 + d4 * d4, zf)

    pltpu.make_async_copy(clsT.at[pl.ds(0, HALF * A), :], clsbuf, semg).wait()
    lab_cp.wait()

    def score_body(m, sc_acc):
        mc = pl.multiple_of(m * C, 16)
        labs = [lab_v[pl.ds(mc + k * L, L)] for k in range(C // L)]
        for a in range(A):
            ma = plsc.load_gather(msk_v, [_splat_i(a * HALF) + m])
            sacc = zf
            for k in range(C // L):
                d = clsbuf[m * A + a, pl.ds(k * L, L)] - labs[k]
                sacc = sacc + d * d
            sc_acc = sc_acc + ma * sacc
        return sc_acc

    score_acc = lax.fori_loop(0, HALF, score_body, zf)

    for cp in plane_cps:
        cp.wait()

    def dense_body(r, dn_acc):
        for k in range(W // L):
            s = _sigmoid(planes_v[r, pl.ds(k * L, L)])
            dn_acc = dn_acc + s * s
        return dn_acc

    dense_acc = lax.fori_loop(0, A * 32, dense_body, zf)

    accs = [corr_acc, obj_acc, prior_acc, true_acc, score_acc, dense_acc, zf, zf]
    for i, v in enumerate(accs):
        acc_v[pl.ds(i * L, L)] = v
    pltpu.sync_copy(acc_v, out_hbm.at[pl.ds(pl.multiple_of(wid * 128, 128), 128)])


@jax.jit
def _yolo_loss_sc(clsT, pred4, predT2, labF, tobF):
    mesh = plsc.VectorSubcoreMesh(core_axis_name="c", subcore_axis_name="s")
    run = functools.partial(
        pl.kernel,
        mesh=mesh,
        compiler_params=pltpu.CompilerParams(
            needs_layout_passes=False, use_tc_tiling_on_sc=True),
        out_type=jax.ShapeDtypeStruct((NW * 128,), jnp.float32),
        scratch_types=[
            pltpu.VMEM((4 * MPAD,), jnp.float32),
            pltpu.VMEM((MPAD,), jnp.int32),
            pltpu.VMEM((2 * HALF,), jnp.int32),
            pltpu.VMEM((HALF,), jnp.int32),
            pltpu.VMEM((HALF,), jnp.int32),
            pltpu.VMEM((HALF * A, C), jnp.float32),
            pltpu.VMEM((HALF, 4 * PP), jnp.float32),
            pltpu.VMEM((A * 32, W), jnp.float32),
            pltpu.VMEM((HALF * C,), jnp.float32),
            pltpu.VMEM((A * HALF,), jnp.float32),
            pltpu.VMEM((8 * L,), jnp.float32),
            pltpu.SemaphoreType.DMA,
            pltpu.SemaphoreType.DMA,
            pltpu.SemaphoreType.DMA,
            pltpu.SemaphoreType.DMA,
        ],
    )(_sc_body)
    return run(clsT, pred4, predT2, labF, tobF)


def kernel(cls_score, pred_object, true_label, true_object, iter_num):
    clsT = jnp.transpose(cls_score, (0, 1, 3, 2, 4)).reshape(CLS_ROWS, C)
    predT2 = jnp.transpose(pred_object, (0, 3, 4, 1, 2)).reshape(PLANES * W, H)
    pred4 = jnp.pad(pred_object.reshape(B * ROWS, A * 5), ((0, 0), (0, PP - A * 5))).reshape(B * ROWS // 4, 4 * PP)
    labF = jnp.pad(true_label, ((0, 0), (0, MPAD - M), (0, 0))).reshape(-1)
    tobF = jnp.pad(jnp.transpose(true_object, (0, 2, 1)), ((0, 0), (0, 0), (0, MPAD - M))).reshape(-1)

    out = _yolo_loss_sc(clsT, pred4, predT2, labF, tobF)
    sums = jnp.sum(out.reshape(NW, 8, L), axis=(0, 2))
    corr, obj_s, prior_s, true_s, score_s, dense_s = (sums[i] for i in range(6))

    n1 = float(B * W * H * A)
    need_prior = (iter_num < ANCHOR_TRAIN_ITERS).astype(jnp.float32)
    noobj_loss = SCALE_NOOBJ * 0.5 * (dense_s - corr) / n1
    obj_loss = SCALE_OBJ * 0.5 * obj_s / n1
    prior_loss = need_prior * SCALE_OBJ * 0.5 * prior_s / (n1 * 2)
    true_loss = SCALE_OBJ * 0.5 * true_s / (n1 * 4)
    score_loss = SCALE_OBJ * 0.5 * score_s / (n1 * C)
    return (noobj_loss + obj_loss + prior_loss + true_loss + score_loss) / 4.0
